# baseline (speedup 1.0000x reference)
import functools

import jax
import jax.numpy as jnp
from jax import lax
from jax.experimental import pallas as pl
from jax.experimental.pallas import tpu as pltpu
from jax.experimental.pallas import tpu_sc as plsc

N1 = 32768
N2 = 8192
F = 256
NSEG = 4
QSEG = N1 // NSEG
PSEG = N2 // NSEG
QBLK = 256

NC = 2
NS = 16
NW = NC * NS
CG = 8
CPS = F // CG
CQ = 256
NCHR = QSEG // CQ


def _dense_body(x_ref, w_ref, s_ref, b_ref, o_ref):
    y = jnp.dot(x_ref[...].astype(jnp.bfloat16),
                w_ref[...].astype(jnp.bfloat16),
                preferred_element_type=jnp.float32)
    o_ref[...] = jnp.maximum(y * s_ref[...] + b_ref[...], 0.0)


def _dense_relu(x, w, s, b):
    n, cin = x.shape
    blk = 512
    return pl.pallas_call(
        _dense_body,
        grid=(n // blk,),
        in_specs=[
            pl.BlockSpec((blk, cin), lambda i: (i, 0)),
            pl.BlockSpec((cin, F), lambda i: (0, 0)),
            pl.BlockSpec((1, F), lambda i: (0, 0)),
            pl.BlockSpec((1, F), lambda i: (0, 0)),
        ],
        out_specs=pl.BlockSpec((blk, F), lambda i: (i, 0)),
        out_shape=jax.ShapeDtypeStruct((n, F), jnp.float32),
    )(x, w, s.reshape(1, F), b.reshape(1, F))


def _knn_body(q_ref, pt_ref, iw_ref):
    q = q_ref[...]
    pt = pt_ref[...]
    dot = lax.dot_general(q.astype(jnp.bfloat16), pt.astype(jnp.bfloat16),
                          (((1,), (0,)), ((), ())),
                          preferred_element_type=jnp.float32)
    qq = jnp.sum(q * q, axis=1, keepdims=True)
    pp = jnp.sum(pt * pt, axis=0, keepdims=True)
    d = qq + pp - 2.0 * dot
    iota = lax.broadcasted_iota(jnp.int32, d.shape, 1)
    big = jnp.float32(3.0e38)
    idxs, dists = [], []
    for _ in range(3):
        m = jnp.min(d, axis=1, keepdims=True)
        am = jnp.min(jnp.where(d == m, iota, PSEG), axis=1, keepdims=True)
        idxs.append(am)
        dists.append(m)
        d = jnp.where(iota == am, big, d)
    dist = jnp.concatenate(dists, axis=1)
    rec = 1.0 / (dist + 1e-8)
    wts = rec / jnp.sum(rec, axis=1, keepdims=True)
    x = jnp.concatenate(
        [idxs[0].astype(jnp.float32), idxs[1].astype(jnp.float32),
         idxs[2].astype(jnp.float32), wts,
         jnp.zeros((QBLK, 2), jnp.float32)], axis=1)
    r = lax.broadcasted_iota(jnp.int32, (QBLK, QBLK), 0)
    c = lax.broadcasted_iota(jnp.int32, (QBLK, QBLK), 1)
    eye = (r == c).astype(jnp.float32)
    xt = lax.dot_general(x, eye, (((0,), (0,)), ((), ())),
                         preferred_element_type=jnp.float32,
                         precision=lax.Precision.HIGHEST)
    iw_ref[...] = xt


def _knn(q_pad, pt_pad):
    nblk = N1 // QBLK
    return pl.pallas_call(
        _knn_body,
        grid=(nblk,),
        in_specs=[
            pl.BlockSpec((QBLK, 8), lambda i: (i, 0)),
            pl.BlockSpec((8, PSEG), lambda i: (0, i // (QSEG // QBLK))),
        ],
        out_specs=pl.BlockSpec((8, QBLK), lambda i: (0, i)),
        out_shape=jax.ShapeDtypeStruct((8, N1), jnp.float32),
    )(q_pad, pt_pad)


def _sc_interp_body(iw_hbm, f2c_hbm, f1c_hbm, out_hbm,
                    table_v, iw_v0, iw_v1, acc_v0, acc_v1,
                    isem0, isem1, fsem0, fsem1):
    wid = lax.axis_index("s") * NC + lax.axis_index("c")
    seg = wid // CG
    cg = wid % CG
    qseg0 = seg * QSEG
    pltpu.sync_copy(f2c_hbm.at[cg, pl.ds(seg * PSEG * CPS, PSEG * CPS)],
                    table_v)
    isems = (isem0, isem1)
    fsems = (fsem0, fsem1)
    iw_bufs = (iw_v0, iw_v1)
    acc_bufs = (acc_v0, acc_v1)

    def start(ci, slot):
        q0 = qseg0 + ci * CQ
        pltpu.async_copy(iw_hbm.at[:, pl.ds(q0, CQ)], iw_bufs[slot],
                         isems[slot])
        pltpu.async_copy(f1c_hbm.at[cg, pl.ds(q0 * CPS, CQ * CPS)],
                         acc_bufs[slot], fsems[slot])

    def wait(ci, slot):
        q0 = qseg0 + ci * CQ
        pltpu.make_async_copy(iw_hbm.at[:, pl.ds(q0, CQ)], iw_bufs[slot],
                              isems[slot]).wait()
        pltpu.make_async_copy(f1c_hbm.at[cg, pl.ds(q0 * CPS, CQ * CPS)],
                              acc_bufs[slot], fsems[slot]).wait()

    def compute(ci, slot):
        iwb = iw_bufs[slot]
        accb = acc_bufs[slot]

        def g_body(g, _):
            base = g * 16
            sl = pl.ds(base, 16)
            qa = (lax.broadcasted_iota(jnp.int32, (16,), 0) + base) * CPS
            i0 = iwb[0, sl].astype(jnp.int32) * CPS
            i1 = iwb[1, sl].astype(jnp.int32) * CPS
            i2 = iwb[2, sl].astype(jnp.int32) * CPS
            w0 = iwb[3, sl]
            w1 = iwb[4, sl]
            w2 = iwb[5, sl]
            for c in range(CPS):
                a = plsc.load_gather(accb, [qa + c])
                a = a + w0 * plsc.load_gather(table_v, [i0 + c])
                a = a + w1 * plsc.load_gather(table_v, [i1 + c])
                a = a + w2 * plsc.load_gather(table_v, [i2 + c])
                plsc.store_scatter(accb, [qa + c], a)
            return 0

        lax.fori_loop(0, CQ // 16, g_body, 0)
        pltpu.sync_copy(accb,
                        out_hbm.at[cg, pl.ds((qseg0 + ci * CQ) * CPS,
                                             CQ * CPS)])

    start(0, 0)

    def pair_body(cp, _):
        e = 2 * cp
        start(e + 1, 1)
        wait(e, 0)
        compute(e, 0)
        nxt = lax.rem(e + 2, NCHR)
        start(nxt, 0)
        wait(e + 1, 1)
        compute(e + 1, 1)
        return 0

    lax.fori_loop(0, NCHR // 2, pair_body, 0)
    wait(0, 0)


def _sc_interp(iw, f2c, f1c):
    mesh = plsc.VectorSubcoreMesh(core_axis_name="c", subcore_axis_name="s")
    return pl.kernel(
        _sc_interp_body,
        out_type=jax.ShapeDtypeStruct((CG, N1 * CPS), jnp.float32),
        mesh=mesh,
        compiler_params=pltpu.CompilerParams(needs_layout_passes=False),
        scratch_types=[
            pltpu.VMEM((PSEG * CPS,), jnp.float32),
            pltpu.VMEM((8, CQ), jnp.float32),
            pltpu.VMEM((8, CQ), jnp.float32),
            pltpu.VMEM((CQ * CPS,), jnp.float32),
            pltpu.VMEM((CQ * CPS,), jnp.float32),
            pltpu.SemaphoreType.DMA,
            pltpu.SemaphoreType.DMA,
            pltpu.SemaphoreType.DMA,
            pltpu.SemaphoreType.DMA,
        ],
    )(iw, f2c, f1c)


def kernel(point_1, feat_1, point_2, feat_2, row_splits_1, row_splits_2,
           W1, b1, g1, be1, m1, v1, W2, b2, g2, be2, m2, v2):
    s1 = g1 / jnp.sqrt(v1 + 1e-5)
    b1f = (b1 - m1) * s1 + be1
    s2 = g2 / jnp.sqrt(v2 + 1e-5)
    b2f = (b2 - m2) * s2 + be2

    f1 = _dense_relu(feat_1, W1, s1, b1f)
    f2 = _dense_relu(feat_2, W2, s2, b2f)

    q_pad = jnp.pad(point_1, ((0, 0), (0, 5)))
    pt_pad = jnp.pad(point_2, ((0, 0), (0, 5))).T
    iw = _knn(q_pad, pt_pad)

    f2c = f2.reshape(N2, CG, CPS).transpose(1, 0, 2).reshape(CG, N2 * CPS)
    f1c = f1.reshape(N1, CG, CPS).transpose(1, 0, 2).reshape(CG, N1 * CPS)
    out3 = _sc_interp(iw, f2c, f1c)
    return out3.reshape(CG, N1, CPS).transpose(1, 0, 2).reshape(N1, F)


# speedup vs baseline: 4.2189x; 4.2189x over previous
import functools

import jax
import jax.numpy as jnp
from jax import lax
from jax.experimental import pallas as pl
from jax.experimental.pallas import tpu as pltpu
from jax.experimental.pallas import tpu_sc as plsc

N1 = 32768
N2 = 8192
F = 256
NSEG = 4
QSEG = N1 // NSEG
PSEG = N2 // NSEG
QBLK = 256

NC = 2
NS = 16
NW = NC * NS
CG = 8
CPS = F // CG
CQ = 256
NCHR = QSEG // CQ


def _eye(n):
    r = lax.broadcasted_iota(jnp.int32, (n, n), 0)
    c = lax.broadcasted_iota(jnp.int32, (n, n), 1)
    return (r == c).astype(jnp.float32)


def _mxu_t(x):
    return lax.dot_general(x, _eye(x.shape[0]), (((0,), (0,)), ((), ())),
                           preferred_element_type=jnp.float32,
                           precision=lax.Precision.HIGHEST)


def _dense_body(x_ref, w_ref, s_ref, b_ref, o_ref):
    y = jnp.dot(x_ref[...].astype(jnp.bfloat16),
                w_ref[...].astype(jnp.bfloat16),
                preferred_element_type=jnp.float32)
    o_ref[...] = jnp.maximum(y * s_ref[...] + b_ref[...], 0.0)


def _dense_t_body(x_ref, w_ref, s_ref, b_ref, o_ref):
    y = jnp.dot(x_ref[...].astype(jnp.bfloat16),
                w_ref[...].astype(jnp.bfloat16),
                preferred_element_type=jnp.float32)
    o_ref[...] = _mxu_t(jnp.maximum(y * s_ref[...] + b_ref[...], 0.0))


def _dense_relu(x, w, s, b, transpose_out=False):
    n, cin = x.shape
    blk = 512
    if transpose_out:
        body = _dense_t_body
        out_spec = pl.BlockSpec((F, blk), lambda i: (0, i))
        out_shape = jax.ShapeDtypeStruct((F, n), jnp.float32)
    else:
        body = _dense_body
        out_spec = pl.BlockSpec((blk, F), lambda i: (i, 0))
        out_shape = jax.ShapeDtypeStruct((n, F), jnp.float32)
    return pl.pallas_call(
        body,
        grid=(n // blk,),
        in_specs=[
            pl.BlockSpec((blk, cin), lambda i: (i, 0)),
            pl.BlockSpec((cin, F), lambda i: (0, 0)),
            pl.BlockSpec((1, F), lambda i: (0, 0)),
            pl.BlockSpec((1, F), lambda i: (0, 0)),
        ],
        out_specs=out_spec,
        out_shape=out_shape,
    )(x, w, s.reshape(1, F), b.reshape(1, F))


def _knn_body(q_ref, pt_ref, iw_ref):
    q = q_ref[...]
    pt = pt_ref[...]
    dot = lax.dot_general(q.astype(jnp.bfloat16), pt.astype(jnp.bfloat16),
                          (((1,), (0,)), ((), ())),
                          preferred_element_type=jnp.float32)
    qq = jnp.sum(q * q, axis=1, keepdims=True)
    pp = jnp.sum(pt * pt, axis=0, keepdims=True)
    d = qq + pp - 2.0 * dot
    iota = lax.broadcasted_iota(jnp.int32, d.shape, 1)
    big = jnp.float32(3.0e38)
    idxs, dists = [], []
    for _ in range(3):
        m = jnp.min(d, axis=1, keepdims=True)
        am = jnp.min(jnp.where(d == m, iota, PSEG), axis=1, keepdims=True)
        idxs.append(am)
        dists.append(m)
        d = jnp.where(iota == am, big, d)
    dist = jnp.concatenate(dists, axis=1)
    rec = 1.0 / (dist + 1e-8)
    wts = rec / jnp.sum(rec, axis=1, keepdims=True)
    x = jnp.concatenate(
        [idxs[0].astype(jnp.float32), idxs[1].astype(jnp.float32),
         idxs[2].astype(jnp.float32), wts,
         jnp.zeros((QBLK, 2), jnp.float32)], axis=1)
    iw_ref[...] = _mxu_t(x)


def _knn(q_pad, pt_pad):
    nblk = N1 // QBLK
    return pl.pallas_call(
        _knn_body,
        grid=(nblk,),
        in_specs=[
            pl.BlockSpec((QBLK, 8), lambda i: (i, 0)),
            pl.BlockSpec((8, PSEG), lambda i: (0, i // (QSEG // QBLK))),
        ],
        out_specs=pl.BlockSpec((8, QBLK), lambda i: (0, i)),
        out_shape=jax.ShapeDtypeStruct((8, N1), jnp.float32),
    )(q_pad, pt_pad)


def _sc_interp_body(iw_hbm, f2t_hbm, out_hbm,
                    table_v, iw_v0, iw_v1, acc_v0, acc_v1, isem0, isem1):
    wid = lax.axis_index("s") * NC + lax.axis_index("c")
    seg = wid // CG
    cg = wid % CG
    qseg0 = seg * QSEG
    pltpu.sync_copy(
        f2t_hbm.at[pl.ds(cg * CPS, CPS), pl.ds(seg * PSEG, PSEG)], table_v)
    isems = (isem0, isem1)
    iw_bufs = (iw_v0, iw_v1)
    acc_bufs = (acc_v0, acc_v1)

    def start(ci, slot):
        q0 = qseg0 + ci * CQ
        pltpu.async_copy(iw_hbm.at[:, pl.ds(q0, CQ)], iw_bufs[slot],
                         isems[slot])

    def wait(ci, slot):
        q0 = qseg0 + ci * CQ
        pltpu.make_async_copy(iw_hbm.at[:, pl.ds(q0, CQ)], iw_bufs[slot],
                              isems[slot]).wait()

    def compute(ci, slot):
        iwb = iw_bufs[slot]
        accb = acc_bufs[slot]

        def g_body(g, _):
            base = g * 16
            sl = pl.ds(base, 16)
            i0 = iwb[0, sl].astype(jnp.int32)
            i1 = iwb[1, sl].astype(jnp.int32)
            i2 = iwb[2, sl].astype(jnp.int32)
            w0 = iwb[3, sl]
            w1 = iwb[4, sl]
            w2 = iwb[5, sl]
            for c in range(CPS):
                cc = jnp.full((16,), c, jnp.int32)
                v = w0 * plsc.load_gather(table_v, [cc, i0])
                v = v + w1 * plsc.load_gather(table_v, [cc, i1])
                v = v + w2 * plsc.load_gather(table_v, [cc, i2])
                accb[c, sl] = v
            return 0

        lax.fori_loop(0, CQ // 16, g_body, 0)
        pltpu.sync_copy(
            accb,
            out_hbm.at[pl.ds(cg * CPS, CPS), pl.ds(qseg0 + ci * CQ, CQ)])

    start(0, 0)

    def pair_body(cp, _):
        e = 2 * cp
        start(e + 1, 1)
        wait(e, 0)
        compute(e, 0)
        nxt = lax.rem(e + 2, NCHR)
        start(nxt, 0)
        wait(e + 1, 1)
        compute(e + 1, 1)
        return 0

    lax.fori_loop(0, NCHR // 2, pair_body, 0)
    wait(0, 0)


def _sc_interp(iw, f2t):
    mesh = plsc.VectorSubcoreMesh(core_axis_name="c", subcore_axis_name="s")
    return pl.kernel(
        _sc_interp_body,
        out_type=jax.ShapeDtypeStruct((F, N1), jnp.float32),
        mesh=mesh,
        compiler_params=pltpu.CompilerParams(needs_layout_passes=False),
        scratch_types=[
            pltpu.VMEM((CPS, PSEG), jnp.float32),
            pltpu.VMEM((8, CQ), jnp.float32),
            pltpu.VMEM((8, CQ), jnp.float32),
            pltpu.VMEM((CPS, CQ), jnp.float32),
            pltpu.VMEM((CPS, CQ), jnp.float32),
            pltpu.SemaphoreType.DMA,
            pltpu.SemaphoreType.DMA,
        ],
    )(iw, f2t)


def _fuse_body(f1_ref, it_ref, o_ref):
    o_ref[...] = f1_ref[...] + _mxu_t(it_ref[...])


def _fuse(f1, interp_t):
    blk = 256
    return pl.pallas_call(
        _fuse_body,
        grid=(N1 // blk,),
        in_specs=[
            pl.BlockSpec((blk, F), lambda i: (i, 0)),
            pl.BlockSpec((F, blk), lambda i: (0, i)),
        ],
        out_specs=pl.BlockSpec((blk, F), lambda i: (i, 0)),
        out_shape=jax.ShapeDtypeStruct((N1, F), jnp.float32),
    )(f1, interp_t)


def kernel(point_1, feat_1, point_2, feat_2, row_splits_1, row_splits_2,
           W1, b1, g1, be1, m1, v1, W2, b2, g2, be2, m2, v2):
    s1 = g1 / jnp.sqrt(v1 + 1e-5)
    b1f = (b1 - m1) * s1 + be1
    s2 = g2 / jnp.sqrt(v2 + 1e-5)
    b2f = (b2 - m2) * s2 + be2

    f1 = _dense_relu(feat_1, W1, s1, b1f)
    f2t = _dense_relu(feat_2, W2, s2, b2f, transpose_out=True)

    q_pad = jnp.pad(point_1, ((0, 0), (0, 5)))
    pt_pad = jnp.pad(point_2, ((0, 0), (0, 5))).T
    iw = _knn(q_pad, pt_pad)

    interp_t = _sc_interp(iw, f2t)
    return _fuse(f1, interp_t)


# speedup vs baseline: 4.4755x; 1.0608x over previous
import functools

import jax
import jax.numpy as jnp
from jax import lax
from jax.experimental import pallas as pl
from jax.experimental.pallas import tpu as pltpu
from jax.experimental.pallas import tpu_sc as plsc

N1 = 32768
N2 = 8192
F = 256
NSEG = 4
QSEG = N1 // NSEG
PSEG = N2 // NSEG
QBLK = 256

NC = 2
NS = 16
NW = NC * NS
CG = 8
CPS = F // CG
CQ = 256
NCHR = QSEG // CQ


def _mxu_t(x):
    return x.T


def _dense_body(x_ref, w_ref, s_ref, b_ref, o_ref):
    y = jnp.dot(x_ref[...].astype(jnp.bfloat16),
                w_ref[...].astype(jnp.bfloat16),
                preferred_element_type=jnp.float32)
    o_ref[...] = jnp.maximum(y * s_ref[...] + b_ref[...], 0.0)


def _dense_t_body(x_ref, w_ref, s_ref, b_ref, o_ref):
    y = jnp.dot(x_ref[...].astype(jnp.bfloat16),
                w_ref[...].astype(jnp.bfloat16),
                preferred_element_type=jnp.float32)
    o_ref[...] = _mxu_t(jnp.maximum(y * s_ref[...] + b_ref[...], 0.0))


def _dense_relu(x, w, s, b, transpose_out=False):
    n, cin = x.shape
    blk = 512
    if transpose_out:
        body = _dense_t_body
        out_spec = pl.BlockSpec((F, blk), lambda i: (0, i))
        out_shape = jax.ShapeDtypeStruct((F, n), jnp.float32)
    else:
        body = _dense_body
        out_spec = pl.BlockSpec((blk, F), lambda i: (i, 0))
        out_shape = jax.ShapeDtypeStruct((n, F), jnp.float32)
    return pl.pallas_call(
        body,
        grid=(n // blk,),
        in_specs=[
            pl.BlockSpec((blk, cin), lambda i: (i, 0)),
            pl.BlockSpec((cin, F), lambda i: (0, 0)),
            pl.BlockSpec((1, F), lambda i: (0, 0)),
            pl.BlockSpec((1, F), lambda i: (0, 0)),
        ],
        out_specs=out_spec,
        out_shape=out_shape,
    )(x, w, s.reshape(1, F), b.reshape(1, F))


def _knn_body(q_ref, pt_ref, iw_ref):
    q = q_ref[...]
    pt = pt_ref[...]
    dot = lax.dot_general(q.astype(jnp.bfloat16), pt.astype(jnp.bfloat16),
                          (((1,), (0,)), ((), ())),
                          preferred_element_type=jnp.float32)
    qq = jnp.sum(q * q, axis=1, keepdims=True)
    pp = jnp.sum(pt * pt, axis=0, keepdims=True)
    d = qq + pp - 2.0 * dot
    iota = lax.broadcasted_iota(jnp.int32, d.shape, 1)
    big = jnp.float32(3.0e38)
    idxs, dists = [], []
    for _ in range(3):
        m = jnp.min(d, axis=1, keepdims=True)
        am = jnp.min(jnp.where(d == m, iota, PSEG), axis=1, keepdims=True)
        idxs.append(am)
        dists.append(m)
        d = jnp.where(iota == am, big, d)
    dist = jnp.concatenate(dists, axis=1)
    rec = 1.0 / (dist + 1e-8)
    wts = rec / jnp.sum(rec, axis=1, keepdims=True)
    x = jnp.concatenate(
        [idxs[0].astype(jnp.float32), idxs[1].astype(jnp.float32),
         idxs[2].astype(jnp.float32), wts,
         jnp.zeros((QBLK, 2), jnp.float32)], axis=1)
    iw_ref[...] = _mxu_t(x)


def _knn(q_pad, pt_pad):
    nblk = N1 // QBLK
    return pl.pallas_call(
        _knn_body,
        grid=(nblk,),
        in_specs=[
            pl.BlockSpec((QBLK, 8), lambda i: (i, 0)),
            pl.BlockSpec((8, PSEG), lambda i: (0, i // (QSEG // QBLK))),
        ],
        out_specs=pl.BlockSpec((8, QBLK), lambda i: (0, i)),
        out_shape=jax.ShapeDtypeStruct((8, N1), jnp.float32),
    )(q_pad, pt_pad)


def _sc_interp_body(iw_hbm, f2t_hbm, out_hbm,
                    table_v, iw_v0, iw_v1, acc_v0, acc_v1, isem0, isem1):
    wid = lax.axis_index("s") * NC + lax.axis_index("c")
    seg = wid // CG
    cg = wid % CG
    qseg0 = seg * QSEG
    pltpu.sync_copy(
        f2t_hbm.at[pl.ds(cg * CPS, CPS), pl.ds(seg * PSEG, PSEG)], table_v)
    isems = (isem0, isem1)
    iw_bufs = (iw_v0, iw_v1)
    acc_bufs = (acc_v0, acc_v1)

    def start(ci, slot):
        q0 = qseg0 + ci * CQ
        pltpu.async_copy(iw_hbm.at[:, pl.ds(q0, CQ)], iw_bufs[slot],
                         isems[slot])

    def wait(ci, slot):
        q0 = qseg0 + ci * CQ
        pltpu.make_async_copy(iw_hbm.at[:, pl.ds(q0, CQ)], iw_bufs[slot],
                              isems[slot]).wait()

    def compute(ci, slot):
        iwb = iw_bufs[slot]
        accb = acc_bufs[slot]

        def g_body(g, _):
            base = g * 16
            sl = pl.ds(base, 16)
            i0 = iwb[0, sl].astype(jnp.int32)
            i1 = iwb[1, sl].astype(jnp.int32)
            i2 = iwb[2, sl].astype(jnp.int32)
            w0 = iwb[3, sl]
            w1 = iwb[4, sl]
            w2 = iwb[5, sl]
            for c in range(CPS):
                cc = jnp.full((16,), c, jnp.int32)
                v = w0 * plsc.load_gather(table_v, [cc, i0])
                v = v + w1 * plsc.load_gather(table_v, [cc, i1])
                v = v + w2 * plsc.load_gather(table_v, [cc, i2])
                accb[c, sl] = v
            return 0

        lax.fori_loop(0, CQ // 16, g_body, 0)
        pltpu.sync_copy(
            accb,
            out_hbm.at[pl.ds(cg * CPS, CPS), pl.ds(qseg0 + ci * CQ, CQ)])

    start(0, 0)

    def pair_body(cp, _):
        e = 2 * cp
        start(e + 1, 1)
        wait(e, 0)
        compute(e, 0)
        nxt = lax.rem(e + 2, NCHR)
        start(nxt, 0)
        wait(e + 1, 1)
        compute(e + 1, 1)
        return 0

    lax.fori_loop(0, NCHR // 2, pair_body, 0)
    wait(0, 0)


def _sc_interp(iw, f2t):
    mesh = plsc.VectorSubcoreMesh(core_axis_name="c", subcore_axis_name="s")
    return pl.kernel(
        _sc_interp_body,
        out_type=jax.ShapeDtypeStruct((F, N1), jnp.float32),
        mesh=mesh,
        compiler_params=pltpu.CompilerParams(needs_layout_passes=False),
        scratch_types=[
            pltpu.VMEM((CPS, PSEG), jnp.float32),
            pltpu.VMEM((8, CQ), jnp.float32),
            pltpu.VMEM((8, CQ), jnp.float32),
            pltpu.VMEM((CPS, CQ), jnp.float32),
            pltpu.VMEM((CPS, CQ), jnp.float32),
            pltpu.SemaphoreType.DMA,
            pltpu.SemaphoreType.DMA,
        ],
    )(iw, f2t)


def _fuse_body(f1_ref, it_ref, o_ref):
    o_ref[...] = f1_ref[...] + _mxu_t(it_ref[...])


def _fuse(f1, interp_t):
    blk = 256
    return pl.pallas_call(
        _fuse_body,
        grid=(N1 // blk,),
        in_specs=[
            pl.BlockSpec((blk, F), lambda i: (i, 0)),
            pl.BlockSpec((F, blk), lambda i: (0, i)),
        ],
        out_specs=pl.BlockSpec((blk, F), lambda i: (i, 0)),
        out_shape=jax.ShapeDtypeStruct((N1, F), jnp.float32),
    )(f1, interp_t)


def kernel(point_1, feat_1, point_2, feat_2, row_splits_1, row_splits_2,
           W1, b1, g1, be1, m1, v1, W2, b2, g2, be2, m2, v2):
    s1 = g1 / jnp.sqrt(v1 + 1e-5)
    b1f = (b1 - m1) * s1 + be1
    s2 = g2 / jnp.sqrt(v2 + 1e-5)
    b2f = (b2 - m2) * s2 + be2

    f1 = _dense_relu(feat_1, W1, s1, b1f)
    f2t = _dense_relu(feat_2, W2, s2, b2f, transpose_out=True)

    q_pad = jnp.pad(point_1, ((0, 0), (0, 5)))
    pt_pad = jnp.pad(point_2, ((0, 0), (0, 5))).T
    iw = _knn(q_pad, pt_pad)

    interp_t = _sc_interp(iw, f2t)
    return _fuse(f1, interp_t)


# speedup vs baseline: 4.6310x; 1.0348x over previous
import functools

import jax
import jax.numpy as jnp
from jax import lax
from jax.experimental import pallas as pl
from jax.experimental.pallas import tpu as pltpu
from jax.experimental.pallas import tpu_sc as plsc

N1 = 32768
N2 = 8192
F = 256
NSEG = 4
QSEG = N1 // NSEG
PSEG = N2 // NSEG
QBLK = 512

NC = 2
NS = 16
NW = NC * NS
CG = 8
CPS = F // CG
CQ = 256
NCHR = QSEG // CQ


def _mxu_t(x):
    return x.T


def _dense_body(x_ref, w_ref, s_ref, b_ref, o_ref):
    y = jnp.dot(x_ref[...].astype(jnp.bfloat16),
                w_ref[...].astype(jnp.bfloat16),
                preferred_element_type=jnp.float32)
    o_ref[...] = jnp.maximum(y * s_ref[...] + b_ref[...], 0.0)


def _dense_t_body(x_ref, w_ref, s_ref, b_ref, o_ref):
    y = jnp.dot(x_ref[...].astype(jnp.bfloat16),
                w_ref[...].astype(jnp.bfloat16),
                preferred_element_type=jnp.float32)
    o_ref[...] = _mxu_t(jnp.maximum(y * s_ref[...] + b_ref[...], 0.0))


def _dense_relu(x, w, s, b, transpose_out=False):
    n, cin = x.shape
    blk = 512
    if transpose_out:
        body = _dense_t_body
        out_spec = pl.BlockSpec((F, blk), lambda i: (0, i))
        out_shape = jax.ShapeDtypeStruct((F, n), jnp.float32)
    else:
        body = _dense_body
        out_spec = pl.BlockSpec((blk, F), lambda i: (i, 0))
        out_shape = jax.ShapeDtypeStruct((n, F), jnp.float32)
    return pl.pallas_call(
        body,
        grid=(n // blk,),
        in_specs=[
            pl.BlockSpec((blk, cin), lambda i: (i, 0)),
            pl.BlockSpec((cin, F), lambda i: (0, 0)),
            pl.BlockSpec((1, F), lambda i: (0, 0)),
            pl.BlockSpec((1, F), lambda i: (0, 0)),
        ],
        out_specs=out_spec,
        out_shape=out_shape,
    )(x, w, s.reshape(1, F), b.reshape(1, F))


def _knn_body(q_ref, pt_ref, iw_ref):
    q = q_ref[...]
    pt = pt_ref[...]
    dot = lax.dot_general(q.astype(jnp.bfloat16), pt.astype(jnp.bfloat16),
                          (((1,), (0,)), ((), ())),
                          preferred_element_type=jnp.float32)
    qq = jnp.sum(q * q, axis=1, keepdims=True)
    pp = jnp.sum(pt * pt, axis=0, keepdims=True)
    d = qq + pp - 2.0 * dot
    iotaf = lax.broadcasted_iota(jnp.int32, d.shape, 1).astype(jnp.float32)
    big = jnp.float32(3.0e38)
    bigi = jnp.float32(PSEG)
    m1 = jnp.min(d, axis=1, keepdims=True)
    am1 = jnp.min(jnp.where(d == m1, iotaf, bigi), axis=1, keepdims=True)
    e1 = iotaf == am1
    m2 = jnp.min(jnp.where(e1, big, d), axis=1, keepdims=True)
    am2 = jnp.min(jnp.where((d == m2) & ~e1, iotaf, bigi),
                  axis=1, keepdims=True)
    e12 = e1 | (iotaf == am2)
    m3 = jnp.min(jnp.where(e12, big, d), axis=1, keepdims=True)
    am3 = jnp.min(jnp.where((d == m3) & ~e12, iotaf, bigi),
                  axis=1, keepdims=True)
    dist = jnp.concatenate([m1, m2, m3], axis=1)
    rec = 1.0 / (dist + 1e-8)
    wts = rec / jnp.sum(rec, axis=1, keepdims=True)
    x = jnp.concatenate(
        [am1, am2, am3, wts,
         jnp.zeros((QBLK, 2), jnp.float32)], axis=1)
    iw_ref[...] = _mxu_t(x)


def _knn(q_pad, pt_pad):
    nblk = N1 // QBLK
    return pl.pallas_call(
        _knn_body,
        grid=(nblk,),
        in_specs=[
            pl.BlockSpec((QBLK, 8), lambda i: (i, 0)),
            pl.BlockSpec((8, PSEG), lambda i: (0, i // (QSEG // QBLK))),
        ],
        out_specs=pl.BlockSpec((8, QBLK), lambda i: (0, i)),
        out_shape=jax.ShapeDtypeStruct((8, N1), jnp.float32),
    )(q_pad, pt_pad)


def _sc_interp_body(iw_hbm, f2t_hbm, out_hbm,
                    table_v, iw_v0, iw_v1, acc_v0, acc_v1, isem0, isem1):
    wid = lax.axis_index("s") * NC + lax.axis_index("c")
    seg = wid // CG
    cg = wid % CG
    qseg0 = seg * QSEG
    pltpu.sync_copy(
        f2t_hbm.at[pl.ds(cg * CPS, CPS), pl.ds(seg * PSEG, PSEG)], table_v)
    isems = (isem0, isem1)
    iw_bufs = (iw_v0, iw_v1)
    acc_bufs = (acc_v0, acc_v1)

    def start(ci, slot):
        q0 = qseg0 + ci * CQ
        pltpu.async_copy(iw_hbm.at[:, pl.ds(q0, CQ)], iw_bufs[slot],
                         isems[slot])

    def wait(ci, slot):
        q0 = qseg0 + ci * CQ
        pltpu.make_async_copy(iw_hbm.at[:, pl.ds(q0, CQ)], iw_bufs[slot],
                              isems[slot]).wait()

    def compute(ci, slot):
        iwb = iw_bufs[slot]
        accb = acc_bufs[slot]

        def g_body(g, _):
            base = g * 16
            sl = pl.ds(base, 16)
            i0 = iwb[0, sl].astype(jnp.int32)
            i1 = iwb[1, sl].astype(jnp.int32)
            i2 = iwb[2, sl].astype(jnp.int32)
            w0 = iwb[3, sl]
            w1 = iwb[4, sl]
            w2 = iwb[5, sl]
            for c in range(CPS):
                cc = jnp.full((16,), c, jnp.int32)
                v = w0 * plsc.load_gather(table_v, [cc, i0])
                v = v + w1 * plsc.load_gather(table_v, [cc, i1])
                v = v + w2 * plsc.load_gather(table_v, [cc, i2])
                accb[c, sl] = v
            return 0

        lax.fori_loop(0, CQ // 16, g_body, 0)
        pltpu.sync_copy(
            accb,
            out_hbm.at[pl.ds(cg * CPS, CPS), pl.ds(qseg0 + ci * CQ, CQ)])

    start(0, 0)

    def pair_body(cp, _):
        e = 2 * cp
        start(e + 1, 1)
        wait(e, 0)
        compute(e, 0)
        nxt = lax.rem(e + 2, NCHR)
        start(nxt, 0)
        wait(e + 1, 1)
        compute(e + 1, 1)
        return 0

    lax.fori_loop(0, NCHR // 2, pair_body, 0)
    wait(0, 0)


def _sc_interp(iw, f2t):
    mesh = plsc.VectorSubcoreMesh(core_axis_name="c", subcore_axis_name="s")
    return pl.kernel(
        _sc_interp_body,
        out_type=jax.ShapeDtypeStruct((F, N1), jnp.float32),
        mesh=mesh,
        compiler_params=pltpu.CompilerParams(needs_layout_passes=False),
        scratch_types=[
            pltpu.VMEM((CPS, PSEG), jnp.float32),
            pltpu.VMEM((8, CQ), jnp.float32),
            pltpu.VMEM((8, CQ), jnp.float32),
            pltpu.VMEM((CPS, CQ), jnp.float32),
            pltpu.VMEM((CPS, CQ), jnp.float32),
            pltpu.SemaphoreType.DMA,
            pltpu.SemaphoreType.DMA,
        ],
    )(iw, f2t)


def _fuse_body(f1_ref, it_ref, o_ref):
    o_ref[...] = f1_ref[...] + _mxu_t(it_ref[...])


def _fuse(f1, interp_t):
    blk = 256
    return pl.pallas_call(
        _fuse_body,
        grid=(N1 // blk,),
        in_specs=[
            pl.BlockSpec((blk, F), lambda i: (i, 0)),
            pl.BlockSpec((F, blk), lambda i: (0, i)),
        ],
        out_specs=pl.BlockSpec((blk, F), lambda i: (i, 0)),
        out_shape=jax.ShapeDtypeStruct((N1, F), jnp.float32),
    )(f1, interp_t)


def kernel(point_1, feat_1, point_2, feat_2, row_splits_1, row_splits_2,
           W1, b1, g1, be1, m1, v1, W2, b2, g2, be2, m2, v2):
    s1 = g1 / jnp.sqrt(v1 + 1e-5)
    b1f = (b1 - m1) * s1 + be1
    s2 = g2 / jnp.sqrt(v2 + 1e-5)
    b2f = (b2 - m2) * s2 + be2

    f1 = _dense_relu(feat_1, W1, s1, b1f)
    f2t = _dense_relu(feat_2, W2, s2, b2f, transpose_out=True)

    q_pad = jnp.pad(point_1, ((0, 0), (0, 5)))
    pt_pad = jnp.pad(point_2, ((0, 0), (0, 5))).T
    iw = _knn(q_pad, pt_pad)

    interp_t = _sc_interp(iw, f2t)
    return _fuse(f1, interp_t)


# speedup vs baseline: 4.6348x; 1.0008x over previous
import jax
import jax.numpy as jnp
from jax import lax
from jax.experimental import pallas as pl
from jax.experimental.pallas import tpu as pltpu
from jax.experimental.pallas import tpu_sc as plsc

N1 = 32768
N2 = 8192
F = 256
NSEG = 4
QSEG = N1 // NSEG
PSEG = N2 // NSEG
QBLK = 512

NC = 2
NS = 16
NW = NC * NS
CG = 8
CPS = F // CG
CQ = 256
NCHR = QSEG // CQ


def _mxu_t(x):
    return x.T


def _dense_body(x_ref, w_ref, s_ref, b_ref, o_ref):
    y = jnp.dot(x_ref[...].astype(jnp.bfloat16),
                w_ref[...].astype(jnp.bfloat16),
                preferred_element_type=jnp.float32)
    o_ref[...] = jnp.maximum(y * s_ref[...] + b_ref[...], 0.0)


def _dense_t_body(x_ref, w_ref, s_ref, b_ref, o_ref):
    y = jnp.dot(x_ref[...].astype(jnp.bfloat16),
                w_ref[...].astype(jnp.bfloat16),
                preferred_element_type=jnp.float32)
    o_ref[...] = _mxu_t(jnp.maximum(y * s_ref[...] + b_ref[...], 0.0))


def _dense_relu(x, w, s, b, transpose_out=False):
    n, cin = x.shape
    blk = 512
    if transpose_out:
        body = _dense_t_body
        out_spec = pl.BlockSpec((F, blk), lambda i: (0, i))
        out_shape = jax.ShapeDtypeStruct((F, n), jnp.float32)
    else:
        body = _dense_body
        out_spec = pl.BlockSpec((blk, F), lambda i: (i, 0))
        out_shape = jax.ShapeDtypeStruct((n, F), jnp.float32)
    return pl.pallas_call(
        body,
        grid=(n // blk,),
        in_specs=[
            pl.BlockSpec((blk, cin), lambda i: (i, 0)),
            pl.BlockSpec((cin, F), lambda i: (0, 0)),
            pl.BlockSpec((1, F), lambda i: (0, 0)),
            pl.BlockSpec((1, F), lambda i: (0, 0)),
        ],
        out_specs=out_spec,
        out_shape=out_shape,
    )(x, w, s.reshape(1, F), b.reshape(1, F))


def _knn_body(q_ref, pt_ref, iw_ref):
    q = q_ref[...]
    pt = pt_ref[...]
    dot = lax.dot_general(q.astype(jnp.bfloat16), pt.astype(jnp.bfloat16),
                          (((1,), (0,)), ((), ())),
                          preferred_element_type=jnp.float32)
    qq = jnp.sum(q * q, axis=1, keepdims=True)
    pp = jnp.sum(pt * pt, axis=0, keepdims=True)
    d = qq + pp - 2.0 * dot
    iotaf = lax.broadcasted_iota(jnp.int32, d.shape, 1).astype(jnp.float32)
    big = jnp.float32(3.0e38)
    bigi = jnp.float32(PSEG)
    m1 = jnp.min(d, axis=1, keepdims=True)
    am1 = jnp.min(jnp.where(d == m1, iotaf, bigi), axis=1, keepdims=True)
    e1 = iotaf == am1
    m2 = jnp.min(jnp.where(e1, big, d), axis=1, keepdims=True)
    am2 = jnp.min(jnp.where((d == m2) & ~e1, iotaf, bigi),
                  axis=1, keepdims=True)
    e12 = e1 | (iotaf == am2)
    m3 = jnp.min(jnp.where(e12, big, d), axis=1, keepdims=True)
    am3 = jnp.min(jnp.where((d == m3) & ~e12, iotaf, bigi),
                  axis=1, keepdims=True)
    dist = jnp.concatenate([m1, m2, m3], axis=1)
    rec = 1.0 / (dist + 1e-8)
    wts = rec / jnp.sum(rec, axis=1, keepdims=True)
    x = jnp.concatenate(
        [am1, am2, am3, wts,
         jnp.zeros((QBLK, 2), jnp.float32)], axis=1)
    iw_ref[...] = _mxu_t(x)


def _knn(q_pad, pt_pad):
    nblk = N1 // QBLK
    return pl.pallas_call(
        _knn_body,
        grid=(nblk,),
        in_specs=[
            pl.BlockSpec((QBLK, 8), lambda i: (i, 0)),
            pl.BlockSpec((8, PSEG), lambda i: (0, i // (QSEG // QBLK))),
        ],
        out_specs=pl.BlockSpec((8, QBLK), lambda i: (0, i)),
        out_shape=jax.ShapeDtypeStruct((8, N1), jnp.float32),
    )(q_pad, pt_pad)


def _sc_interp_body(iw_hbm, f2t_hbm, out_hbm,
                    table_v, iw_v0, iw_v1, acc_v0, acc_v1, isem0, isem1):
    wid = lax.axis_index("s") * NC + lax.axis_index("c")
    seg = wid // CG
    cg = wid % CG
    qseg0 = seg * QSEG
    pltpu.sync_copy(
        f2t_hbm.at[pl.ds(cg * CPS, CPS), pl.ds(seg * PSEG, PSEG)], table_v)
    isems = (isem0, isem1)
    iw_bufs = (iw_v0, iw_v1)
    acc_bufs = (acc_v0, acc_v1)

    def start(ci, slot):
        q0 = qseg0 + ci * CQ
        pltpu.async_copy(iw_hbm.at[:, pl.ds(q0, CQ)], iw_bufs[slot],
                         isems[slot])

    def wait(ci, slot):
        q0 = qseg0 + ci * CQ
        pltpu.make_async_copy(iw_hbm.at[:, pl.ds(q0, CQ)], iw_bufs[slot],
                              isems[slot]).wait()

    def compute(ci, slot):
        iwb = iw_bufs[slot]
        accb = acc_bufs[slot]

        def g_body(g, _):
            base = g * 16
            sl = pl.ds(base, 16)
            i0 = iwb[0, sl].astype(jnp.int32)
            i1 = iwb[1, sl].astype(jnp.int32)
            i2 = iwb[2, sl].astype(jnp.int32)
            w0 = iwb[3, sl]
            w1 = iwb[4, sl]
            w2 = iwb[5, sl]
            for c in range(CPS):
                cc = jnp.full((16,), c, jnp.int32)
                v = w0 * plsc.load_gather(table_v, [cc, i0])
                v = v + w1 * plsc.load_gather(table_v, [cc, i1])
                v = v + w2 * plsc.load_gather(table_v, [cc, i2])
                accb[c, sl] = v
            return 0

        lax.fori_loop(0, CQ // 16, g_body, 0)
        pltpu.sync_copy(
            accb,
            out_hbm.at[pl.ds(cg * CPS, CPS), pl.ds(qseg0 + ci * CQ, CQ)])

    start(0, 0)

    def pair_body(cp, _):
        e = 2 * cp
        start(e + 1, 1)
        wait(e, 0)
        compute(e, 0)
        nxt = lax.rem(e + 2, NCHR)
        start(nxt, 0)
        wait(e + 1, 1)
        compute(e + 1, 1)
        return 0

    lax.fori_loop(0, NCHR // 2, pair_body, 0)
    wait(0, 0)


def _sc_interp(iw, f2t):
    mesh = plsc.VectorSubcoreMesh(core_axis_name="c", subcore_axis_name="s")
    return pl.kernel(
        _sc_interp_body,
        out_type=jax.ShapeDtypeStruct((F, N1), jnp.float32),
        mesh=mesh,
        compiler_params=pltpu.CompilerParams(needs_layout_passes=False),
        scratch_types=[
            pltpu.VMEM((CPS, PSEG), jnp.float32),
            pltpu.VMEM((8, CQ), jnp.float32),
            pltpu.VMEM((8, CQ), jnp.float32),
            pltpu.VMEM((CPS, CQ), jnp.float32),
            pltpu.VMEM((CPS, CQ), jnp.float32),
            pltpu.SemaphoreType.DMA,
            pltpu.SemaphoreType.DMA,
        ],
    )(iw, f2t)


def _fuse_body(f1_ref, it_ref, o_ref):
    o_ref[...] = f1_ref[...] + _mxu_t(it_ref[...])


def _fuse(f1, interp_t):
    blk = 256
    return pl.pallas_call(
        _fuse_body,
        grid=(N1 // blk,),
        in_specs=[
            pl.BlockSpec((blk, F), lambda i: (i, 0)),
            pl.BlockSpec((F, blk), lambda i: (0, i)),
        ],
        out_specs=pl.BlockSpec((blk, F), lambda i: (i, 0)),
        out_shape=jax.ShapeDtypeStruct((N1, F), jnp.float32),
    )(f1, interp_t)


def kernel(point_1, feat_1, point_2, feat_2, row_splits_1, row_splits_2,
           W1, b1, g1, be1, m1, v1, W2, b2, g2, be2, m2, v2):
    s1 = g1 / jnp.sqrt(v1 + 1e-5)
    b1f = (b1 - m1) * s1 + be1
    s2 = g2 / jnp.sqrt(v2 + 1e-5)
    b2f = (b2 - m2) * s2 + be2

    f1 = _dense_relu(feat_1, W1, s1, b1f)
    f2t = _dense_relu(feat_2, W2, s2, b2f, transpose_out=True)

    q_pad = jnp.pad(point_1, ((0, 0), (0, 5)))
    pt_pad = jnp.pad(point_2, ((0, 0), (0, 5))).T
    iw = _knn(q_pad, pt_pad)

    interp_t = _sc_interp(iw, f2t)
    return _fuse(f1, interp_t)


# speedup vs baseline: 4.7872x; 1.0329x over previous
import jax
import jax.numpy as jnp
from jax import lax
from jax.experimental import pallas as pl
from jax.experimental.pallas import tpu as pltpu
from jax.experimental.pallas import tpu_sc as plsc

N1 = 32768
N2 = 8192
F = 256
NSEG = 4
QSEG = N1 // NSEG
PSEG = N2 // NSEG
QBLK = 1024

NC = 2
NS = 16
NW = NC * NS
CG = 8
CPS = F // CG
CQ = 256
NCHR = QSEG // CQ


def _mxu_t(x):
    return x.T


def _dense_body(x_ref, w_ref, s_ref, b_ref, o_ref):
    y = jnp.dot(x_ref[...].astype(jnp.bfloat16),
                w_ref[...].astype(jnp.bfloat16),
                preferred_element_type=jnp.float32)
    o_ref[...] = jnp.maximum(y * s_ref[...] + b_ref[...], 0.0)


def _dense_t_body(x_ref, w_ref, s_ref, b_ref, o_ref):
    y = jnp.dot(x_ref[...].astype(jnp.bfloat16),
                w_ref[...].astype(jnp.bfloat16),
                preferred_element_type=jnp.float32)
    o_ref[...] = _mxu_t(jnp.maximum(y * s_ref[...] + b_ref[...], 0.0))


def _dense_relu(x, w, s, b, transpose_out=False):
    n, cin = x.shape
    blk = 512
    if transpose_out:
        body = _dense_t_body
        out_spec = pl.BlockSpec((F, blk), lambda i: (0, i))
        out_shape = jax.ShapeDtypeStruct((F, n), jnp.float32)
    else:
        body = _dense_body
        out_spec = pl.BlockSpec((blk, F), lambda i: (i, 0))
        out_shape = jax.ShapeDtypeStruct((n, F), jnp.float32)
    return pl.pallas_call(
        body,
        grid=(n // blk,),
        in_specs=[
            pl.BlockSpec((blk, cin), lambda i: (i, 0)),
            pl.BlockSpec((cin, F), lambda i: (0, 0)),
            pl.BlockSpec((1, F), lambda i: (0, 0)),
            pl.BlockSpec((1, F), lambda i: (0, 0)),
        ],
        out_specs=out_spec,
        out_shape=out_shape,
    )(x, w, s.reshape(1, F), b.reshape(1, F))


def _knn_body(q_ref, pt_ref, iw_ref):
    q = q_ref[...]
    pt = pt_ref[...]
    dot = lax.dot_general(q.astype(jnp.bfloat16), pt.astype(jnp.bfloat16),
                          (((1,), (0,)), ((), ())),
                          preferred_element_type=jnp.float32)
    qq = jnp.sum(q * q, axis=1, keepdims=True)
    pp = jnp.sum(pt * pt, axis=0, keepdims=True)
    d = qq + pp - 2.0 * dot
    iotaf = lax.broadcasted_iota(jnp.int32, d.shape, 1).astype(jnp.float32)
    big = jnp.float32(3.0e38)
    bigi = jnp.float32(PSEG)
    m1 = jnp.min(d, axis=1, keepdims=True)
    am1 = jnp.min(jnp.where(d == m1, iotaf, bigi), axis=1, keepdims=True)
    e1 = iotaf == am1
    m2 = jnp.min(jnp.where(e1, big, d), axis=1, keepdims=True)
    am2 = jnp.min(jnp.where((d == m2) & ~e1, iotaf, bigi),
                  axis=1, keepdims=True)
    e12 = e1 | (iotaf == am2)
    m3 = jnp.min(jnp.where(e12, big, d), axis=1, keepdims=True)
    am3 = jnp.min(jnp.where((d == m3) & ~e12, iotaf, bigi),
                  axis=1, keepdims=True)
    dist = jnp.concatenate([m1, m2, m3], axis=1)
    rec = 1.0 / (dist + 1e-8)
    wts = rec / jnp.sum(rec, axis=1, keepdims=True)
    x = jnp.concatenate(
        [am1, am2, am3, wts,
         jnp.zeros((QBLK, 2), jnp.float32)], axis=1)
    iw_ref[...] = _mxu_t(x)


def _knn(q_pad, pt_pad):
    nblk = N1 // QBLK
    return pl.pallas_call(
        _knn_body,
        grid=(nblk,),
        in_specs=[
            pl.BlockSpec((QBLK, 8), lambda i: (i, 0)),
            pl.BlockSpec((8, PSEG), lambda i: (0, i // (QSEG // QBLK))),
        ],
        out_specs=pl.BlockSpec((8, QBLK), lambda i: (0, i)),
        out_shape=jax.ShapeDtypeStruct((8, N1), jnp.float32),
    )(q_pad, pt_pad)


def _sc_interp_body(iw_hbm, f2t_hbm, out_hbm,
                    table_v, iw_v0, iw_v1, acc_v0, acc_v1, isem0, isem1):
    wid = lax.axis_index("s") * NC + lax.axis_index("c")
    seg = wid // CG
    cg = wid % CG
    qseg0 = seg * QSEG
    pltpu.sync_copy(
        f2t_hbm.at[pl.ds(cg * CPS, CPS), pl.ds(seg * PSEG, PSEG)], table_v)
    isems = (isem0, isem1)
    iw_bufs = (iw_v0, iw_v1)
    acc_bufs = (acc_v0, acc_v1)

    def start(ci, slot):
        q0 = qseg0 + ci * CQ
        pltpu.async_copy(iw_hbm.at[:, pl.ds(q0, CQ)], iw_bufs[slot],
                         isems[slot])

    def wait(ci, slot):
        q0 = qseg0 + ci * CQ
        pltpu.make_async_copy(iw_hbm.at[:, pl.ds(q0, CQ)], iw_bufs[slot],
                              isems[slot]).wait()

    def compute(ci, slot):
        iwb = iw_bufs[slot]
        accb = acc_bufs[slot]

        def g_body(g, _):
            base = g * 16
            sl = pl.ds(base, 16)
            i0 = iwb[0, sl].astype(jnp.int32)
            i1 = iwb[1, sl].astype(jnp.int32)
            i2 = iwb[2, sl].astype(jnp.int32)
            w0 = iwb[3, sl]
            w1 = iwb[4, sl]
            w2 = iwb[5, sl]
            for c in range(CPS):
                cc = jnp.full((16,), c, jnp.int32)
                v = w0 * plsc.load_gather(table_v, [cc, i0])
                v = v + w1 * plsc.load_gather(table_v, [cc, i1])
                v = v + w2 * plsc.load_gather(table_v, [cc, i2])
                accb[c, sl] = v
            return 0

        lax.fori_loop(0, CQ // 16, g_body, 0)
        pltpu.sync_copy(
            accb,
            out_hbm.at[pl.ds(cg * CPS, CPS), pl.ds(qseg0 + ci * CQ, CQ)])

    start(0, 0)

    def pair_body(cp, _):
        e = 2 * cp
        start(e + 1, 1)
        wait(e, 0)
        compute(e, 0)
        nxt = lax.rem(e + 2, NCHR)
        start(nxt, 0)
        wait(e + 1, 1)
        compute(e + 1, 1)
        return 0

    lax.fori_loop(0, NCHR // 2, pair_body, 0)
    wait(0, 0)


def _sc_interp(iw, f2t):
    mesh = plsc.VectorSubcoreMesh(core_axis_name="c", subcore_axis_name="s")
    return pl.kernel(
        _sc_interp_body,
        out_type=jax.ShapeDtypeStruct((F, N1), jnp.float32),
        mesh=mesh,
        compiler_params=pltpu.CompilerParams(needs_layout_passes=False),
        scratch_types=[
            pltpu.VMEM((CPS, PSEG), jnp.float32),
            pltpu.VMEM((8, CQ), jnp.float32),
            pltpu.VMEM((8, CQ), jnp.float32),
            pltpu.VMEM((CPS, CQ), jnp.float32),
            pltpu.VMEM((CPS, CQ), jnp.float32),
            pltpu.SemaphoreType.DMA,
            pltpu.SemaphoreType.DMA,
        ],
    )(iw, f2t)


def _fuse_body(f1_ref, it_ref, o_ref):
    o_ref[...] = f1_ref[...] + _mxu_t(it_ref[...])


def _fuse(f1, interp_t):
    blk = 256
    return pl.pallas_call(
        _fuse_body,
        grid=(N1 // blk,),
        in_specs=[
            pl.BlockSpec((blk, F), lambda i: (i, 0)),
            pl.BlockSpec((F, blk), lambda i: (0, i)),
        ],
        out_specs=pl.BlockSpec((blk, F), lambda i: (i, 0)),
        out_shape=jax.ShapeDtypeStruct((N1, F), jnp.float32),
    )(f1, interp_t)


def kernel(point_1, feat_1, point_2, feat_2, row_splits_1, row_splits_2,
           W1, b1, g1, be1, m1, v1, W2, b2, g2, be2, m2, v2):
    s1 = g1 / jnp.sqrt(v1 + 1e-5)
    b1f = (b1 - m1) * s1 + be1
    s2 = g2 / jnp.sqrt(v2 + 1e-5)
    b2f = (b2 - m2) * s2 + be2

    f1 = _dense_relu(feat_1, W1, s1, b1f)
    f2t = _dense_relu(feat_2, W2, s2, b2f, transpose_out=True)

    q_pad = jnp.pad(point_1, ((0, 0), (0, 5)))
    pt_pad = jnp.pad(point_2, ((0, 0), (0, 5))).T
    iw = _knn(q_pad, pt_pad)

    interp_t = _sc_interp(iw, f2t)
    return _fuse(f1, interp_t)
